# Initial kernel scaffold; baseline (speedup 1.0000x reference)
import jax
import jax.numpy as jnp
from jax.experimental import pallas as pl


def kernel(x, edge_index, batch, Wproj, Wl, Wr, att, b, ln_w, ln_b, Wh, bh):
    raise NotImplementedError("write your pallas kernel here")


# speedup vs baseline: 10.4700x; 10.4700x over previous
import functools

import jax
import jax.numpy as jnp
from jax import lax
from jax.experimental import pallas as pl
from jax.experimental.pallas import tpu as pltpu
from jax.experimental.pallas import tpu_sc as plsc

NS = 16
NCORE = 2
K = 32
EPI_R = 32
ACC_W = 144


def _proj_body(x_ref, w_ref, o_ref):
    o_ref[...] = jnp.dot(x_ref[...], w_ref[...],
                         preferred_element_type=jnp.float32)


def _dual_body(h_ref, wl_ref, wr_ref, xl_ref, xr_ref):
    h = h_ref[...]
    xl_ref[...] = jnp.dot(h, wl_ref[...], preferred_element_type=jnp.float32)
    xr_ref[...] = jnp.dot(h, wr_ref[...], preferred_element_type=jnp.float32)


def _post_body(p_ref, b_ref, w_ref, lb_ref, o_ref):
    n = o_ref.shape[0]
    ar = p_ref.shape[0] // 2
    o = 0.25 * (p_ref[0:n, :] + p_ref[ar:ar + n, :]) + b_ref[...]
    m = jnp.mean(o)
    xm = o - m
    v = jnp.mean(xm * xm)
    o = xm / (jnp.sqrt(v) + 1e-5) * w_ref[...] + lb_ref[...]
    o_ref[...] = 1.0507009873554805 * jnp.where(
        o > 0, o, 1.6732632423543772 * (jnp.exp(o) - 1.0))


def _pool_body(h_ref, wh_ref, bh_ref, o_ref):
    n, c = h_ref.shape
    def body(i, acc):
        return acc + h_ref[pl.ds(i * 8, 8), :]
    acc = lax.fori_loop(0, n // 8, body, jnp.zeros((8, c), jnp.float32))
    r = acc[0:1, :]
    for i in range(1, 8):
        r = r + acc[i:i + 1, :]
    o_ref[...] = (jnp.sum((r / n) * wh_ref[...], keepdims=True)
                  + bh_ref[...])


def _tc_proj(x, w, interpret=False):
    n = x.shape[0]
    return pl.pallas_call(
        _proj_body,
        out_shape=jax.ShapeDtypeStruct((n, w.shape[1]), jnp.float32),
        interpret=interpret,
    )(x, w)


def _tc_dual(h, wl, wr, interpret=False):
    n, c = h.shape
    hc2 = wl.shape[1]
    half = hc2 // 2
    rb = 2000
    nb = n // rb
    return pl.pallas_call(
        _dual_body,
        grid=(2, nb),
        in_specs=[
            pl.BlockSpec((rb, c), lambda i, j: (j, 0)),
            pl.BlockSpec((c, half), lambda i, j: (0, i)),
            pl.BlockSpec((c, half), lambda i, j: (0, i)),
        ],
        out_specs=[
            pl.BlockSpec((rb, half), lambda i, j: (i * nb + j, 0)),
            pl.BlockSpec((rb, half), lambda i, j: (i * nb + j, 0)),
        ],
        out_shape=[
            jax.ShapeDtypeStruct((2 * n, half), jnp.float32),
            jax.ShapeDtypeStruct((2 * n, half), jnp.float32),
        ],
        interpret=interpret,
    )(h, wl, wr)


def _tc_post(p, b, w, lb, n, interpret=False):
    c = p.shape[1]
    return pl.pallas_call(
        _post_body,
        out_shape=jax.ShapeDtypeStruct((n, c), jnp.float32),
        interpret=interpret,
    )(p, b.reshape(1, c), w.reshape(1, c), lb.reshape(1, c))


def _tc_pool(h, wh, bh, interpret=False):
    return pl.pallas_call(
        _pool_body,
        out_shape=jax.ShapeDtypeStruct((1, 1), jnp.float32),
        interpret=interpret,
    )(h, wh.reshape(1, -1), bh.reshape(1, 1))


@functools.lru_cache(maxsize=None)
def _build_sc_edge(np_, ep, interpret=False):
    te = ep // NS
    nch = te // K
    acc_rows = ((np_ + NS * K - 1) // (NS * K)) * (NS * K)
    zr = acc_rows // NS
    dsz = 2 * acc_rows
    mesh = plsc.VectorSubcoreMesh(core_axis_name="c", subcore_axis_name="s",
                                  num_cores=NCORE, num_subcores=NS)

    @functools.partial(
        pl.kernel,
        out_type=[jax.ShapeDtypeStruct((2 * acc_rows, 64), jnp.float32),
                  jax.ShapeDtypeStruct((NCORE * NS * 2 * acc_rows,),
                                       jnp.float32)],
        mesh=mesh,
        interpret=interpret,
        compiler_params=pltpu.CompilerParams(needs_layout_passes=False),
        scratch_types=[
            pltpu.VMEM((K,), jnp.int32),
            pltpu.VMEM((K,), jnp.int32),
            pltpu.VMEM((K,), jnp.int32),
            pltpu.VMEM((K, 128), jnp.float32),
            pltpu.VMEM((K, 128), jnp.float32),
            pltpu.VMEM((K, 128), jnp.float32),
            pltpu.VMEM((128,), jnp.float32),
            pltpu.VMEM((dsz,), jnp.float32),
            pltpu.VMEM((zr,), jnp.float32),
            pltpu.VMEM((zr,), jnp.float32),
            pltpu.VMEM((zr,), jnp.float32),
            pltpu.VMEM((zr,), jnp.float32),
            pltpu.VMEM((EPI_R, 128), jnp.float32),
            pltpu.VMEM((EPI_R, 64), jnp.float32),
            pltpu.VMEM_SHARED((acc_rows, 128), jnp.float32),
            pltpu.SemaphoreType.DMA,
            pltpu.SemaphoreType.DMA,
        ],
    )
    def sc_edge(xl_hbm, xr_hbm, srcg, dstg, dsts, attp, out_hbm, dout_hbm,
                sidx, didx, dsidx, xlb, xrb, msg, attv, dpart,
                dsum0, dsum1, dbuf0, dbuf1, nbuf, obuf,
                acc, sem1, sem2):
        c = lax.axis_index("c")
        s = lax.axis_index("s")
        lane = lax.iota(jnp.int32, 16)
        rots = [(lane + k) & 15 for k in (8, 4, 2, 1)]
        zv = jnp.zeros((16,), jnp.float32)

        def lanesum(v):
            for r in rots:
                v = v + v.at[r].get(mode="promise_in_bounds")
            return v

        pltpu.sync_copy(attp.at[pl.ds(c * 128, 128)], attv)

        def zmsg(i, carry):
            msg[i // 8, pl.ds((i % 8) * 16, 16)] = zv
            return carry
        lax.fori_loop(0, K * 8, zmsg, 0)

        def zdp(i, carry):
            dpart[pl.ds(i * 16, 16)] = zv
            return carry
        lax.fori_loop(0, dsz // 16, zdp, 0)

        def zacc(t, carry):
            pltpu.sync_copy(msg, acc.at[pl.ds(s * zr + t * K, K)])
            return carry
        lax.fori_loop(0, zr // K, zacc, 0)

        plsc.subcore_barrier()

        ebase = c * ep

        def chunk(nc, carry):
            eb = s * te + nc * K
            pltpu.sync_copy(srcg.at[pl.ds(ebase + eb, K)], sidx)
            pltpu.sync_copy(dstg.at[pl.ds(ebase + eb, K)], didx)
            pltpu.sync_copy(dsts.at[pl.ds(eb, K)], dsidx)
            cp1 = pltpu.async_copy(xl_hbm.at[sidx], xlb, sem1)
            cp2 = pltpu.async_copy(xr_hbm.at[didx], xrb, sem2)
            cp1.wait()
            cp2.wait()
            dva = dsidx[pl.ds(0, 16)]
            dvb = dsidx[pl.ds(16, 16)]

            def edge(e, carry2):
                a0 = zv
                a1 = zv
                for j in range(8):
                    z = xrb[e, pl.ds(j * 16, 16)] + xlb[e, pl.ds(j * 16, 16)]
                    zl = jnp.maximum(z, 0.0) + 0.2 * jnp.minimum(z, 0.0)
                    t = zl * attv[pl.ds(j * 16, 16)]
                    if j < 4:
                        a0 = a0 + t
                    else:
                        a1 = a1 + t
                w0 = jnp.exp(lanesum(a0))
                w1 = jnp.exp(lanesum(a1))
                for j in range(4):
                    msg[e, pl.ds(j * 16, 16)] = xlb[e, pl.ds(j * 16, 16)] * w0
                for j in range(4, 8):
                    msg[e, pl.ds(j * 16, 16)] = xlb[e, pl.ds(j * 16, 16)] * w1
                m = lane == (e & 15)
                dv = jnp.where(e < 16, dva, dvb)
                plsc.addupdate_scatter(dpart, [dv], w0, mask=m)
                plsc.addupdate_scatter(dpart, [dv + acc_rows], w1, mask=m)
                return carry2
            lax.fori_loop(0, K, edge, 0)
            pltpu.sync_copy(msg, acc.at[dsidx], add=True)
            return carry
        lax.fori_loop(0, nch, chunk, 0)

        pltpu.sync_copy(dpart, dout_hbm.at[pl.ds((c * NS + s) * dsz, dsz)])
        plsc.subcore_barrier()

        def zds(i, carry):
            dsum0[pl.ds(i * 16, 16)] = zv
            dsum1[pl.ds(i * 16, 16)] = zv
            return carry
        lax.fori_loop(0, zr // 16, zds, 0)

        def dred(t, carry):
            tb = (c * NS + t) * dsz
            pltpu.sync_copy(dout_hbm.at[pl.ds(tb + s * zr, zr)], dbuf0)
            pltpu.sync_copy(dout_hbm.at[pl.ds(tb + acc_rows + s * zr, zr)],
                            dbuf1)

            def dadd(i, carry2):
                sl = pl.ds(i * 16, 16)
                dsum0[sl] = dsum0[sl] + dbuf0[sl]
                dsum1[sl] = dsum1[sl] + dbuf1[sl]
                return carry2
            lax.fori_loop(0, zr // 16, dadd, 0)
            return carry
        lax.fori_loop(0, NS, dred, 0)

        def epi(t, carry):
            r0 = s * zr + t * EPI_R
            pltpu.sync_copy(acc.at[pl.ds(r0, EPI_R)], nbuf)

            def grp(g, carry2):
                q = t * EPI_R + g * 16
                iv0 = 1.0 / (dsum0[pl.ds(q, 16)] + 1e-16)
                iv1 = 1.0 / (dsum1[pl.ds(q, 16)] + 1e-16)
                def nrow(e, carry3):
                    r = g * 16 + e
                    ev = jnp.full((16,), e, jnp.int32)
                    b0 = iv0.at[ev].get(mode="promise_in_bounds")
                    b1 = iv1.at[ev].get(mode="promise_in_bounds")
                    for j in range(4):
                        obuf[r, pl.ds(j * 16, 16)] = (
                            nbuf[r, pl.ds(j * 16, 16)] * b0
                            + nbuf[r, pl.ds(64 + j * 16, 16)] * b1)
                    return carry3
                lax.fori_loop(0, 16, nrow, 0)
                return carry2
            lax.fori_loop(0, EPI_R // 16, grp, 0)
            pltpu.sync_copy(obuf, out_hbm.at[pl.ds(c * acc_rows + r0, EPI_R)])
            return carry
        lax.fori_loop(0, zr // EPI_R, epi, 0)

    return sc_edge


def kernel(x, edge_index, batch, Wproj, Wl, Wr, att, b, ln_w, ln_b, Wh, bh):
    n = x.shape[0]
    e = edge_index.shape[1]
    num_l = Wl.shape[0]
    e2 = e + n
    ep = ((e2 + NS * K - 1) // (NS * K)) * (NS * K)
    pad = ep - e2
    acc_rows = ((n + NS * K - 1) // (NS * K)) * (NS * K)

    loops = jnp.arange(n, dtype=edge_index.dtype)
    src = jnp.concatenate([edge_index[0], loops,
                           jnp.zeros((pad,), edge_index.dtype)])
    dstg = jnp.concatenate([edge_index[1], loops,
                            jnp.zeros((pad,), edge_index.dtype)])
    dsts = jnp.concatenate([edge_index[1], loops,
                            jnp.full((pad,), n, edge_index.dtype)])
    srcg2 = jnp.concatenate([src, src + n])
    dstg2 = jnp.concatenate([dstg, dstg + n])

    sc_edge = _build_sc_edge(n, ep)

    h = _tc_proj(x, Wproj)
    for l in range(num_l):
        xl2, xr2 = _tc_dual(h, Wl[l], Wr[l])
        p, _ = sc_edge(xl2, xr2, srcg2, dstg2, dsts, att[l].reshape(-1))
        h = _tc_post(p, b[l], ln_w[l], ln_b[l], n)
    out = _tc_pool(h, Wh, bh)
    return out.reshape(1)


# speedup vs baseline: 16.1243x; 1.5401x over previous
import functools

import jax
import jax.numpy as jnp
from jax import lax
from jax.experimental import pallas as pl
from jax.experimental.pallas import tpu as pltpu
from jax.experimental.pallas import tpu_sc as plsc

NS = 16
NCORE = 2
K = 32
EPI_R = 16
ACC_W = 144


def _proj_body(x_ref, w_ref, o_ref):
    o_ref[...] = jnp.dot(x_ref[...], w_ref[...],
                         preferred_element_type=jnp.float32)


def _dual_body(h_ref, wl_ref, wr_ref, xl_ref, xr_ref):
    h = h_ref[...]
    xl_ref[...] = jnp.dot(h, wl_ref[...], preferred_element_type=jnp.float32)
    xr_ref[...] = jnp.dot(h, wr_ref[...], preferred_element_type=jnp.float32)


def _post_body(p_ref, b_ref, w_ref, lb_ref, o_ref):
    n = o_ref.shape[0]
    ar = p_ref.shape[0] // 2
    o = 0.25 * (p_ref[0:n, :] + p_ref[ar:ar + n, :]) + b_ref[...]
    m = jnp.mean(o)
    xm = o - m
    v = jnp.mean(xm * xm)
    o = xm / (jnp.sqrt(v) + 1e-5) * w_ref[...] + lb_ref[...]
    o_ref[...] = 1.0507009873554805 * jnp.where(
        o > 0, o, 1.6732632423543772 * (jnp.exp(o) - 1.0))


def _pool_body(h_ref, wh_ref, bh_ref, o_ref):
    n, c = h_ref.shape
    def body(i, acc):
        return acc + h_ref[pl.ds(i * 8, 8), :]
    acc = lax.fori_loop(0, n // 8, body, jnp.zeros((8, c), jnp.float32))
    r = acc[0:1, :]
    for i in range(1, 8):
        r = r + acc[i:i + 1, :]
    o_ref[...] = (jnp.sum((r / n) * wh_ref[...], keepdims=True)
                  + bh_ref[...])


def _tc_proj(x, w, interpret=False):
    n = x.shape[0]
    return pl.pallas_call(
        _proj_body,
        out_shape=jax.ShapeDtypeStruct((n, w.shape[1]), jnp.float32),
        interpret=interpret,
    )(x, w)


def _tc_dual(h, wl, wr, interpret=False):
    n, c = h.shape
    hc2 = wl.shape[1]
    half = hc2 // 2
    rb = 2000
    nb = n // rb
    return pl.pallas_call(
        _dual_body,
        grid=(2, nb),
        in_specs=[
            pl.BlockSpec((rb, c), lambda i, j: (j, 0)),
            pl.BlockSpec((c, half), lambda i, j: (0, i)),
            pl.BlockSpec((c, half), lambda i, j: (0, i)),
        ],
        out_specs=[
            pl.BlockSpec((rb, half), lambda i, j: (i * nb + j, 0)),
            pl.BlockSpec((rb, half), lambda i, j: (i * nb + j, 0)),
        ],
        out_shape=[
            jax.ShapeDtypeStruct((2 * n, half), jnp.float32),
            jax.ShapeDtypeStruct((2 * n, half), jnp.float32),
        ],
        interpret=interpret,
    )(h, wl, wr)


def _tc_post(p, b, w, lb, n, interpret=False):
    c = p.shape[1]
    return pl.pallas_call(
        _post_body,
        out_shape=jax.ShapeDtypeStruct((n, c), jnp.float32),
        interpret=interpret,
    )(p, b.reshape(1, c), w.reshape(1, c), lb.reshape(1, c))


def _tc_pool(h, wh, bh, interpret=False):
    return pl.pallas_call(
        _pool_body,
        out_shape=jax.ShapeDtypeStruct((1, 1), jnp.float32),
        interpret=interpret,
    )(h, wh.reshape(1, -1), bh.reshape(1, 1))


@functools.lru_cache(maxsize=None)
def _build_sc_edge(np_, ep, interpret=False):
    te = ep // NS
    nch = te // K
    acc_rows = ((np_ + NS * K - 1) // (NS * K)) * (NS * K)
    zr = acc_rows // NS
    dsz = 2 * acc_rows
    mesh = plsc.VectorSubcoreMesh(core_axis_name="c", subcore_axis_name="s",
                                  num_cores=NCORE, num_subcores=NS)

    @functools.partial(
        pl.kernel,
        out_type=[jax.ShapeDtypeStruct((2 * acc_rows, 64), jnp.float32),
                  jax.ShapeDtypeStruct((NCORE * NS * 2 * acc_rows,),
                                       jnp.float32)],
        mesh=mesh,
        interpret=interpret,
        compiler_params=pltpu.CompilerParams(needs_layout_passes=False),
        scratch_types=[
            pltpu.VMEM((K,), jnp.int32),
            pltpu.VMEM((K,), jnp.int32),
            pltpu.VMEM((K,), jnp.int32),
            pltpu.VMEM((K,), jnp.int32),
            pltpu.VMEM((K,), jnp.int32),
            pltpu.VMEM((K,), jnp.int32),
            pltpu.VMEM((K, 128), jnp.float32),
            pltpu.VMEM((K, 128), jnp.float32),
            pltpu.VMEM((K, 128), jnp.float32),
            pltpu.VMEM((K, 128), jnp.float32),
            pltpu.VMEM((128,), jnp.float32),
            pltpu.VMEM((dsz,), jnp.float32),
            pltpu.VMEM((zr,), jnp.float32),
            pltpu.VMEM((zr,), jnp.float32),
            pltpu.VMEM((zr,), jnp.float32),
            pltpu.VMEM((zr,), jnp.float32),
            pltpu.VMEM((EPI_R, 128), jnp.float32),
            pltpu.VMEM((EPI_R, 64), jnp.float32),
            pltpu.VMEM_SHARED((acc_rows, 128), jnp.float32),
            pltpu.SemaphoreType.DMA,
            pltpu.SemaphoreType.DMA,
            pltpu.SemaphoreType.DMA,
            pltpu.SemaphoreType.DMA,
        ],
    )
    def sc_edge(xl_hbm, xr_hbm, srcg, dstg, dsts, attp, out_hbm, dout_hbm,
                sidx0, sidx1, didx0, didx1, dsidx0, dsidx1,
                xlb0, xlb1, xrb0, xrb1, attv, dpart,
                dsum0, dsum1, dbuf0, dbuf1, nbuf, obuf,
                acc, semxl0, semxl1, semxr0, semxr1):
        c = lax.axis_index("c")
        s = lax.axis_index("s")
        lane = lax.iota(jnp.int32, 16)
        rots = [(lane + k) & 15 for k in (8, 4, 2, 1)]
        zv = jnp.zeros((16,), jnp.float32)
        sidx = (sidx0, sidx1)
        didx = (didx0, didx1)
        dsidx = (dsidx0, dsidx1)
        xlb = (xlb0, xlb1)
        xrb = (xrb0, xrb1)
        semxl = (semxl0, semxl1)
        semxr = (semxr0, semxr1)

        def lanesum(v):
            for r in rots:
                v = v + v.at[r].get(mode="promise_in_bounds")
            return v

        pltpu.sync_copy(attp.at[pl.ds(c * 128, 128)], attv)

        def zmsg(i, carry):
            xlb0[i // 8, pl.ds((i % 8) * 16, 16)] = zv
            return carry
        lax.fori_loop(0, K * 8, zmsg, 0)

        def zdp(i, carry):
            dpart[pl.ds(i * 16, 16)] = zv
            return carry
        lax.fori_loop(0, dsz // 16, zdp, 0)

        def zacc(t, carry):
            pltpu.sync_copy(xlb0, acc.at[pl.ds(s * zr + t * K, K)])
            return carry
        lax.fori_loop(0, zr // K, zacc, 0)

        plsc.subcore_barrier()

        ebase = c * ep

        def stage(b, nc):
            eb = s * te + nc * K
            pltpu.sync_copy(srcg.at[pl.ds(ebase + eb, K)], sidx[b])
            pltpu.sync_copy(dstg.at[pl.ds(ebase + eb, K)], didx[b])
            pltpu.sync_copy(dsts.at[pl.ds(eb, K)], dsidx[b])
            pltpu.async_copy(xl_hbm.at[sidx[b]], xlb[b], semxl[b])
            pltpu.async_copy(xr_hbm.at[didx[b]], xrb[b], semxr[b])

        stage(0, 0)

        def pair(g, carry):
            for b in range(2):
                nc = g * 2 + b
                nb = 1 - b
                pltpu.make_async_copy(xl_hbm.at[sidx[b]], xlb[b],
                                      semxl[b]).wait()
                pltpu.make_async_copy(xr_hbm.at[didx[b]], xrb[b],
                                      semxr[b]).wait()

                @pl.when(nc + 1 < nch)
                def _():
                    stage(nb, nc + 1)

                dva = dsidx[b][pl.ds(0, 16)]
                dvb = dsidx[b][pl.ds(16, 16)]

                def edge(e, carry2, _b=b, _dva=dva, _dvb=dvb):
                    xb = xlb[_b]
                    yb = xrb[_b]
                    a0 = zv
                    a1 = zv
                    for j in range(8):
                        z = yb[e, pl.ds(j * 16, 16)] + xb[e, pl.ds(j * 16, 16)]
                        zl = jnp.maximum(z, 0.0) + 0.2 * jnp.minimum(z, 0.0)
                        t = zl * attv[pl.ds(j * 16, 16)]
                        if j < 4:
                            a0 = a0 + t
                        else:
                            a1 = a1 + t
                    w0 = jnp.exp(lanesum(a0))
                    w1 = jnp.exp(lanesum(a1))
                    for j in range(4):
                        xb[e, pl.ds(j * 16, 16)] = xb[e, pl.ds(j * 16, 16)] * w0
                    for j in range(4, 8):
                        xb[e, pl.ds(j * 16, 16)] = xb[e, pl.ds(j * 16, 16)] * w1
                    m = lane == (e & 15)
                    dv = jnp.where(e < 16, _dva, _dvb)
                    plsc.addupdate_scatter(dpart, [dv], w0, mask=m)
                    plsc.addupdate_scatter(dpart, [dv + acc_rows], w1, mask=m)
                    return carry2
                lax.fori_loop(0, K, edge, 0)
                pltpu.sync_copy(xlb[b], acc.at[dsidx[b]], add=True)
            return carry
        lax.fori_loop(0, nch // 2, pair, 0)

        pltpu.sync_copy(dpart, dout_hbm.at[pl.ds((c * NS + s) * dsz, dsz)])
        plsc.subcore_barrier()

        def zds(i, carry):
            dsum0[pl.ds(i * 16, 16)] = zv
            dsum1[pl.ds(i * 16, 16)] = zv
            return carry
        lax.fori_loop(0, zr // 16, zds, 0)

        def dred(t, carry):
            tb = (c * NS + t) * dsz
            pltpu.sync_copy(dout_hbm.at[pl.ds(tb + s * zr, zr)], dbuf0)
            pltpu.sync_copy(dout_hbm.at[pl.ds(tb + acc_rows + s * zr, zr)],
                            dbuf1)

            def dadd(i, carry2):
                sl = pl.ds(i * 16, 16)
                dsum0[sl] = dsum0[sl] + dbuf0[sl]
                dsum1[sl] = dsum1[sl] + dbuf1[sl]
                return carry2
            lax.fori_loop(0, zr // 16, dadd, 0)
            return carry
        lax.fori_loop(0, NS, dred, 0)

        def epi(t, carry):
            r0 = s * zr + t * EPI_R
            pltpu.sync_copy(acc.at[pl.ds(r0, EPI_R)], nbuf)

            def grp(g, carry2):
                q = t * EPI_R + g * 16
                iv0 = 1.0 / (dsum0[pl.ds(q, 16)] + 1e-16)
                iv1 = 1.0 / (dsum1[pl.ds(q, 16)] + 1e-16)

                def nrow(e, carry3):
                    r = g * 16 + e
                    ev = jnp.full((16,), e, jnp.int32)
                    b0 = iv0.at[ev].get(mode="promise_in_bounds")
                    b1 = iv1.at[ev].get(mode="promise_in_bounds")
                    for j in range(4):
                        obuf[r, pl.ds(j * 16, 16)] = (
                            nbuf[r, pl.ds(j * 16, 16)] * b0
                            + nbuf[r, pl.ds(64 + j * 16, 16)] * b1)
                    return carry3
                lax.fori_loop(0, 16, nrow, 0)
                return carry2
            lax.fori_loop(0, EPI_R // 16, grp, 0)
            pltpu.sync_copy(obuf, out_hbm.at[pl.ds(c * acc_rows + r0, EPI_R)])
            return carry
        lax.fori_loop(0, zr // EPI_R, epi, 0)

    return sc_edge


def kernel(x, edge_index, batch, Wproj, Wl, Wr, att, b, ln_w, ln_b, Wh, bh):
    n = x.shape[0]
    e = edge_index.shape[1]
    num_l = Wl.shape[0]
    e2 = e + n
    ep = ((e2 + 2 * NS * K - 1) // (2 * NS * K)) * (2 * NS * K)
    pad = ep - e2
    acc_rows = ((n + NS * K - 1) // (NS * K)) * (NS * K)

    loops = jnp.arange(n, dtype=edge_index.dtype)
    src = jnp.concatenate([edge_index[0], loops,
                           jnp.zeros((pad,), edge_index.dtype)])
    dstg = jnp.concatenate([edge_index[1], loops,
                            jnp.zeros((pad,), edge_index.dtype)])
    dsts = jnp.concatenate([edge_index[1], loops,
                            jnp.full((pad,), n, edge_index.dtype)])
    srcg2 = jnp.concatenate([src, src + n])
    dstg2 = jnp.concatenate([dstg, dstg + n])

    sc_edge = _build_sc_edge(n, ep)

    h = _tc_proj(x, Wproj)
    for l in range(num_l):
        xl2, xr2 = _tc_dual(h, Wl[l], Wr[l])
        p, _ = sc_edge(xl2, xr2, srcg2, dstg2, dsts, att[l].reshape(-1))
        h = _tc_post(p, b[l], ln_w[l], ln_b[l], n)
    out = _tc_pool(h, Wh, bh)
    return out.reshape(1)
